# baseline (speedup 1.0000x reference)
import functools

import jax
import jax.numpy as jnp
from jax import lax
from jax.experimental import pallas as pl
from jax.experimental.pallas import tpu as pltpu
from jax.experimental.pallas import tpu_sc as plsc

B, M, N = 8, 1024, 256
H, W = 128, 128
HW = H * W
NCH = 2
CHUNKS = 32
HALF = 32


def _sc_body(x_hbm, locy_hbm, locx_hbm, out_hbm, yv, xv, idxv, maskv,
             dirty_m, dirty_p, xband, buf0, buf1, dsem0, dsem1):
    c = lax.axis_index("c")
    s = lax.axis_index("s")
    wid = c * 16 + s
    b = wid // 4
    band = wid % 4

    pltpu.sync_copy(locy_hbm.at[b], yv)
    pltpu.sync_copy(locx_hbm.at[b], xv)

    def idx_body(g, carry):
        ys = yv[pl.ds(g * 16, 16)]
        xcs = xv[pl.ds(g * 16, 16)]
        idxv[pl.ds(g * 16, 16)] = ys * W + xcs
        return carry

    lax.fori_loop(0, M // 16, idx_body, 0)

    lanes = lax.iota(jnp.int32, 16)
    m4 = lanes < NCH
    sct_base = jnp.where(m4, lanes * HW, 0)
    zeros16 = jnp.zeros((16,), jnp.float32)

    def dup_body(g, nd):
        pv = idxv[pl.ds(g * 16, 16)]
        _, last = plsc.scan_count(pv)
        maskv[pl.ds(g * 16, 16)] = jnp.where(last, -1, 0).astype(jnp.int32)
        dup = ~last
        k = plsc.all_reduce_population_count(dup)[0]
        plsc.store_compressed(dirty_m.at[pl.ds(nd, 16)], g * 16 + lanes,
                              mask=dup)
        plsc.store_compressed(dirty_p.at[pl.ds(nd, 16)], pv, mask=dup)
        return nd + k

    ndirty = lax.fori_loop(0, M // 16, dup_body, 0)

    @plsc.parallel_loop(0, NCH * HW // 16, unroll=16)
    def _zero(i):
        buf0[pl.ds(i * 16, 16)] = zeros16
        buf1[pl.ds(i * 16, 16)] = zeros16

    def half_body(hf, carry):
        pltpu.sync_copy(
            x_hbm.at[b, :, pl.ds(band * 64 + hf * HALF, HALF)], xband)

        def super_body(t2, carry1):
            for k, buf, dsem in ((0, buf0, dsem0), (1, buf1, dsem1)):
                ch_i = hf * (CHUNKS // 2) + t2 * 2 + k
                cg = band * CHUNKS + ch_i
                base = (b * N + cg * NCH) * HW
                lc0 = (t2 * 2 + k) * NCH

                @pl.when((t2 > 0) | (hf > 0))
                def _():
                    pltpu.make_async_copy(
                        buf, out_hbm.at[pl.ds(0, NCH * HW)], dsem).wait()

                @plsc.parallel_loop(0, M // 16, unroll=8)
                def _rezero(g):
                    pv = idxv[pl.ds(g * 16, 16)]
                    for ch in range(NCH):
                        plsc.store_scatter(buf, [pv + ch * HW], zeros16)

                @plsc.parallel_loop(0, M // 16, unroll=4)
                def _accum(g):
                    lastm = maskv[pl.ds(g * 16, 16)] != 0
                    pv = idxv[pl.ds(g * 16, 16)]
                    mvec = g * 16 + lanes
                    for ch in range(NCH):
                        vals = plsc.load_gather(
                            xband, [mvec, jnp.full((16,), lc0 + ch,
                                                   jnp.int32)])
                        plsc.addupdate_scatter(buf, [pv + ch * HW], vals,
                                               mask=lastm)

                chv = jnp.where(m4, lc0 + lanes, 0)

                @pl.when(ndirty > 0)
                def _():
                    def dirty_body(blk, carry2):
                        dmv = dirty_m[pl.ds(blk * 16, 16)]
                        dpv = dirty_p[pl.ds(blk * 16, 16)]
                        for j in range(16):
                            ok = blk * 16 + j < ndirty
                            mj = m4 & ok
                            vals = plsc.load_gather(
                                xband, [jnp.full((16,), dmv[j], jnp.int32),
                                        chv], mask=mj)
                            plsc.addupdate_scatter(
                                buf, [sct_base + dpv[j]], vals, mask=mj)
                        return carry2

                    lax.fori_loop(0, (ndirty + 15) // 16, dirty_body, 0)

                pltpu.async_copy(buf, out_hbm.at[pl.ds(base, NCH * HW)], dsem)
            return carry1

        lax.fori_loop(0, CHUNKS // 4, super_body, 0)
        return carry

    lax.fori_loop(0, 2, half_body, 0)

    pltpu.make_async_copy(buf0, out_hbm.at[pl.ds(0, NCH * HW)], dsem0).wait()
    pltpu.make_async_copy(buf1, out_hbm.at[pl.ds(0, NCH * HW)], dsem1).wait()


def kernel(x, spatial_size, location):
    del spatial_size
    loc = location.astype(jnp.int32)
    locy = loc[:, :, 0]
    locx = loc[:, :, 1]

    sc = functools.partial(
        pl.kernel,
        out_type=jax.ShapeDtypeStruct((B * N * HW,), jnp.float32),
        mesh=plsc.VectorSubcoreMesh(core_axis_name="c", subcore_axis_name="s"),
        compiler_params=pltpu.CompilerParams(needs_layout_passes=False,
                                             use_tc_tiling_on_sc=False),
        scratch_types=[
            pltpu.VMEM((M,), jnp.int32),
            pltpu.VMEM((M,), jnp.int32),
            pltpu.VMEM((M,), jnp.int32),
            pltpu.VMEM((M,), jnp.int32),
            pltpu.VMEM((M + 16,), jnp.int32),
            pltpu.VMEM((M + 16,), jnp.int32),
            pltpu.VMEM((M, HALF), jnp.float32),
            pltpu.VMEM((NCH * HW,), jnp.float32),
            pltpu.VMEM((NCH * HW,), jnp.float32),
            pltpu.SemaphoreType.DMA,
            pltpu.SemaphoreType.DMA,
        ],
    )(_sc_body)
    out = sc(x, locy, locx)
    return out.reshape(B, N, H, W)


# speedup vs baseline: 1.2199x; 1.2199x over previous
import functools

import jax
import jax.numpy as jnp
from jax import lax
from jax.experimental import pallas as pl
from jax.experimental.pallas import tpu as pltpu
from jax.experimental.pallas import tpu_sc as plsc

B, M, N = 8, 1024, 256
H, W = 128, 128
HW = H * W
NCH = 2
CHUNKS = 32


def _sc_body(xt_hbm, locy_hbm, locx_hbm, out_hbm, yv, xv, idxv, maskv,
             dirty_m, dirty_p, xs0, xs1, buf0, buf1, xsem0, xsem1, dsem0,
             dsem1):
    c = lax.axis_index("c")
    s = lax.axis_index("s")
    wid = c * 16 + s
    b = wid // 4
    band = wid % 4

    pltpu.sync_copy(locy_hbm.at[b], yv)
    pltpu.sync_copy(locx_hbm.at[b], xv)

    def idx_body(g, carry):
        ys = yv[pl.ds(g * 16, 16)]
        xcs = xv[pl.ds(g * 16, 16)]
        idxv[pl.ds(g * 16, 16)] = ys * W + xcs
        return carry

    lax.fori_loop(0, M // 16, idx_body, 0)

    lanes = lax.iota(jnp.int32, 16)
    m4 = lanes < NCH
    gat_base = jnp.where(m4, lanes * M, 0)
    sct_base = jnp.where(m4, lanes * HW, 0)
    zeros16 = jnp.zeros((16,), jnp.float32)

    def dup_body(g, nd):
        pv = idxv[pl.ds(g * 16, 16)]
        _, last = plsc.scan_count(pv)
        maskv[pl.ds(g * 16, 16)] = jnp.where(last, -1, 0).astype(jnp.int32)
        dup = ~last
        k = plsc.all_reduce_population_count(dup)[0]
        plsc.store_compressed(dirty_m.at[pl.ds(nd, 16)], g * 16 + lanes,
                              mask=dup)
        plsc.store_compressed(dirty_p.at[pl.ds(nd, 16)], pv, mask=dup)
        return nd + k

    ndirty = lax.fori_loop(0, M // 16, dup_body, 0)

    @plsc.parallel_loop(0, NCH * HW // 16, unroll=16)
    def _zero(i):
        buf0[pl.ds(i * 16, 16)] = zeros16
        buf1[pl.ds(i * 16, 16)] = zeros16

    pltpu.async_copy(xt_hbm.at[b, pl.ds(band * CHUNKS * NCH * M, NCH * M)],
                     xs0, xsem0)

    def super_body(t2, carry):
        for k, xs, buf, xsem, dsem, oxs, oxsem in (
                (0, xs0, buf0, xsem0, dsem0, xs1, xsem1),
                (1, xs1, buf1, xsem1, dsem1, xs0, xsem0)):
            ch_i = t2 * 2 + k
            cg = band * CHUNKS + ch_i
            base = (b * N + cg * NCH) * HW

            nxt = band * CHUNKS + jnp.minimum(ch_i + 1, CHUNKS - 1)
            pltpu.async_copy(xt_hbm.at[b, pl.ds(nxt * NCH * M, NCH * M)],
                             oxs, oxsem)

            @pl.when(t2 > 0)
            def _():
                pltpu.make_async_copy(
                    buf, out_hbm.at[pl.ds(0, NCH * HW)], dsem).wait()

            @plsc.parallel_loop(0, M // 16, unroll=8)
            def _rezero(g):
                pv = idxv[pl.ds(g * 16, 16)]
                for ch in range(NCH):
                    plsc.store_scatter(buf, [pv + ch * HW], zeros16)

            pltpu.make_async_copy(xt_hbm.at[b, pl.ds(0, NCH * M)], xs,
                                  xsem).wait()

            @plsc.parallel_loop(0, M // 16, unroll=4)
            def _accum(g):
                lastm = maskv[pl.ds(g * 16, 16)] != 0
                pv = idxv[pl.ds(g * 16, 16)]
                for ch in range(NCH):
                    vals = xs[pl.ds(ch * M + g * 16, 16)]
                    plsc.addupdate_scatter(buf, [pv + ch * HW], vals,
                                           mask=lastm)

            @pl.when(ndirty > 0)
            def _():
                def dirty_body(blk, carry2):
                    dmv = dirty_m[pl.ds(blk * 16, 16)]
                    dpv = dirty_p[pl.ds(blk * 16, 16)]
                    for j in range(16):
                        ok = blk * 16 + j < ndirty
                        mj = m4 & ok
                        vals = plsc.load_gather(
                            xs, [gat_base + dmv[j]], mask=mj)
                        plsc.addupdate_scatter(
                            buf, [sct_base + dpv[j]], vals, mask=mj)
                    return carry2

                lax.fori_loop(0, (ndirty + 15) // 16, dirty_body, 0)

            pltpu.async_copy(buf, out_hbm.at[pl.ds(base, NCH * HW)], dsem)
        return carry

    lax.fori_loop(0, CHUNKS // 2, super_body, 0)

    pltpu.make_async_copy(buf0, out_hbm.at[pl.ds(0, NCH * HW)], dsem0).wait()
    pltpu.make_async_copy(buf1, out_hbm.at[pl.ds(0, NCH * HW)], dsem1).wait()
    pltpu.make_async_copy(xt_hbm.at[b, pl.ds(0, NCH * M)], xs0, xsem0).wait()


def kernel(x, spatial_size, location):
    del spatial_size
    loc = location.astype(jnp.int32)
    locy = loc[:, :, 0]
    locx = loc[:, :, 1]
    xt = jnp.transpose(x, (0, 2, 1)).reshape(B, N * M)

    sc = functools.partial(
        pl.kernel,
        out_type=jax.ShapeDtypeStruct((B * N * HW,), jnp.float32),
        mesh=plsc.VectorSubcoreMesh(core_axis_name="c", subcore_axis_name="s"),
        compiler_params=pltpu.CompilerParams(needs_layout_passes=False),
        scratch_types=[
            pltpu.VMEM((M,), jnp.int32),
            pltpu.VMEM((M,), jnp.int32),
            pltpu.VMEM((M,), jnp.int32),
            pltpu.VMEM((M,), jnp.int32),
            pltpu.VMEM((M + 16,), jnp.int32),
            pltpu.VMEM((M + 16,), jnp.int32),
            pltpu.VMEM((NCH * M,), jnp.float32),
            pltpu.VMEM((NCH * M,), jnp.float32),
            pltpu.VMEM((NCH * HW,), jnp.float32),
            pltpu.VMEM((NCH * HW,), jnp.float32),
            pltpu.SemaphoreType.DMA,
            pltpu.SemaphoreType.DMA,
            pltpu.SemaphoreType.DMA,
            pltpu.SemaphoreType.DMA,
        ],
    )(_sc_body)
    out = sc(xt, locy, locx)
    return out.reshape(B, N, H, W)
